# baseline (speedup 1.0000x reference)
import functools

import jax
import jax.numpy as jnp
from jax import lax
from jax.experimental import pallas as pl
from jax.experimental.pallas import tpu as pltpu
from jax.experimental.pallas import tpu_sc as plsc

NC = 2
NS = 16
NW = NC * NS
B = 128
NBLK = 512


def _deg_body(rpw, src_hbm, dst_hbm, ones_hbm, zeros_hbm, out_hbm,
              sidx, didx, ones_v, dego_sp, degi_sp, sem_s):
    c = lax.axis_index("c")
    s = lax.axis_index("s")
    wid = c * NS + s
    npad = dego_sp.shape[0]
    sl = npad // NS
    pltpu.sync_copy(zeros_hbm, dego_sp.at[pl.ds(s * sl, sl)])
    pltpu.sync_copy(zeros_hbm, degi_sp.at[pl.ds(s * sl, sl)])
    pltpu.sync_copy(ones_hbm, ones_v)
    base = wid * rpw
    pltpu.sync_copy(src_hbm.at[pl.ds(base, rpw)], sidx)
    pltpu.sync_copy(dst_hbm.at[pl.ds(base, rpw)], didx)
    plsc.subcore_barrier()

    def fire(r, _):
        pltpu.async_copy(ones_v, dego_sp.at[sidx.at[r]], sem_s, add=True)
        pltpu.async_copy(ones_v, degi_sp.at[didx.at[r]], sem_s, add=True)
        return _

    lax.fori_loop(0, rpw, fire, None)

    def drain(r, _):
        pltpu.make_async_copy(ones_hbm, ones_v, sem_s).wait()
        return _

    lax.fori_loop(0, 2 * rpw, drain, None)
    plsc.subcore_barrier()
    pltpu.sync_copy(dego_sp.at[pl.ds(s * sl, sl)],
                    out_hbm.at[c, 0, pl.ds(s * sl, sl)])
    pltpu.sync_copy(degi_sp.at[pl.ds(s * sl, sl)],
                    out_hbm.at[c, 1, pl.ds(s * sl, sl)])


B2 = 32
NSLOT = 8
LEAD = NSLOT - 1


def _agg_body(rpw, src_hbm, dst_hbm, y1_hbm, ndst_hbm, zrow_hbm, zeros_hbm,
              agg_out, c_out,
              sidx, didx, rowsb, vbuf, agg_sp, c_sp, ndst_sp,
              sem_i, sem_g, sem_s, sem_v, sem_sc):
    c = lax.axis_index("c")
    s = lax.axis_index("s")
    wid = c * NS + s
    npad = c_sp.shape[0]
    sl = npad // NS
    nchunks = rpw // 8
    pltpu.sync_copy(zrow_hbm, agg_sp.at[pl.ds(s * sl, sl), :])
    pltpu.sync_copy(zeros_hbm, c_sp.at[pl.ds(s * sl, sl)])
    pltpu.sync_copy(ndst_hbm.at[pl.ds(s * sl, sl)],
                    ndst_sp.at[pl.ds(s * sl, sl)])
    base = wid * rpw

    def stage(q):
        qs = q % 3
        pltpu.async_copy(src_hbm.at[pl.ds(base + q * 8, 8)], sidx.at[qs],
                         sem_i.at[qs])
        pltpu.async_copy(dst_hbm.at[pl.ds(base + q * 8, 8)], didx.at[qs],
                         sem_i.at[qs])

    def wait_stage(q):
        qs = q % 3
        pltpu.make_async_copy(src_hbm.at[pl.ds(0, 8)], sidx.at[qs],
                              sem_i.at[qs]).wait()
        pltpu.make_async_copy(src_hbm.at[pl.ds(0, 8)], didx.at[qs],
                              sem_i.at[qs]).wait()

    def fire_g(r, slot):
        pltpu.async_copy(y1_hbm.at[sidx.at[(r // 8) % 3, r % 8]],
                         rowsb.at[slot], sem_g.at[slot])

    def fire_v(r, slot):
        pltpu.async_copy(ndst_sp.at[didx.at[(r // 8) % 3, r % 8]],
                         vbuf.at[slot], sem_v.at[slot])

    def fire_s(r, slot):
        pltpu.async_copy(rowsb.at[slot],
                         agg_sp.at[didx.at[(r // 8) % 3, r % 8]],
                         sem_s.at[slot], add=True)

    def fire_c(r, slot):
        pltpu.async_copy(vbuf.at[slot], c_sp.at[sidx.at[(r // 8) % 3, r % 8]],
                         sem_sc.at[slot], add=True)

    def wait_rows(sem, slot):
        pltpu.make_async_copy(y1_hbm.at[pl.ds(0, B2)], rowsb.at[slot],
                              sem.at[slot]).wait()

    def wait_vals(sem, slot):
        pltpu.make_async_copy(ndst_hbm.at[pl.ds(0, B2)], vbuf.at[slot],
                              sem.at[slot]).wait()

    stage(0)
    wait_stage(0)
    stage(1)
    plsc.subcore_barrier()
    for g in range(LEAD):
        fire_g(g, g)
    fire_v(0, 0)

    def body(r, _):
        slot = r % NSLOT
        vslot = r % 2
        wait_rows(sem_g, slot)
        fire_s(r, slot)

        @pl.when(r >= 1)
        def _():
            wait_rows(sem_s, (r + LEAD) % NSLOT)

        qn = (r + LEAD) // 8

        @pl.when(((r + LEAD) % 8 == 0) & (qn <= nchunks - 1))
        def _():
            wait_stage(qn)

        @pl.when(((r + LEAD) % 8 == 0) & (qn <= nchunks - 2))
        def _():
            stage(qn + 1)

        @pl.when(r <= rpw - NSLOT)
        def _():
            fire_g(r + LEAD, (r + LEAD) % NSLOT)

        wait_vals(sem_v, vslot)
        fire_c(r, vslot)

        @pl.when(r >= 1)
        def _():
            wait_vals(sem_sc, (r + 1) % 2)

        @pl.when(r < rpw - 1)
        def _():
            fire_v(r + 1, (r + 1) % 2)

        return _

    lax.fori_loop(0, rpw, body, None)
    wait_rows(sem_s, (rpw - 1) % NSLOT)
    wait_vals(sem_sc, (rpw - 1) % 2)
    plsc.subcore_barrier()
    pltpu.sync_copy(agg_sp.at[pl.ds(s * sl, sl), :],
                    agg_out.at[c, pl.ds(s * sl, sl), :])
    pltpu.sync_copy(c_sp.at[pl.ds(s * sl, sl)],
                    c_out.at[c, pl.ds(s * sl, sl)])


def _lin_body(f1_ref, f2_ref, w1_ref, deg_ref, y1_ref, ndst_ref):
    x1 = f1_ref[...]
    x2 = f2_ref[...]
    w = w1_ref[...]
    k = x1.shape[1]
    y0 = jnp.dot(x1, w[:k, :], preferred_element_type=jnp.float32)
    y0 = y0 + jnp.dot(x2, w[k:, :], preferred_element_type=jnp.float32)
    dsrc = deg_ref[0, 0, :] + deg_ref[1, 0, :]
    ddst = deg_ref[0, 1, :] + deg_ref[1, 1, :]
    ns = lax.rsqrt(jnp.maximum(dsrc, 1.0))
    nd = lax.rsqrt(jnp.maximum(ddst, 1.0))
    n = y0.shape[0]
    ii = lax.broadcasted_iota(jnp.int32, (n, n), 0)
    jj = lax.broadcasted_iota(jnp.int32, (n, n), 1)
    dmat = jnp.where(ii == jj, ns[None, :], 0.0)
    y1_ref[...] = jnp.dot(dmat, y0, preferred_element_type=jnp.float32)
    ndst_ref[...] = nd


def _fin_body(nblocks, n_nodes, agg_ref, c_ref, deg_ref, w2_ref, b1_ref,
              b2_ref, out_ref, acc):
    i = pl.program_id(0)
    agg = agg_ref[0] + agg_ref[1]
    ddst = deg_ref[0, 1, :] + deg_ref[1, 1, :]
    dsrc = deg_ref[0, 0, :] + deg_ref[1, 0, :]
    nd = lax.rsqrt(jnp.maximum(ddst, 1.0))
    ns = lax.rsqrt(jnp.maximum(dsrc, 1.0))
    n = agg.shape[0]
    ii = lax.broadcasted_iota(jnp.int32, (n, n), 0)
    jj = lax.broadcasted_iota(jnp.int32, (n, n), 1)
    dmat = jnp.where(ii == jj, nd[None, :], 0.0)
    h = jnp.dot(dmat, agg, preferred_element_type=jnp.float32)
    h = jnp.maximum(h + b1_ref[...][None, :], 0.0)
    node = i * n + lax.broadcasted_iota(jnp.int32, (n,), 0)
    wv = (c_ref[0] + c_ref[1]) * ns
    wv = jnp.where(node < n_nodes, wv, 0.0)
    pv = jnp.dot(wv[None, :], h, preferred_element_type=jnp.float32)

    @pl.when(i == 0)
    def _():
        acc[...] = jnp.zeros_like(acc)

    acc[...] += pv

    @pl.when(i == nblocks - 1)
    def _():
        o = jnp.dot(acc[...], w2_ref[...],
                    preferred_element_type=jnp.float32)
        o = o * (1.0 / n_nodes) + b2_ref[...][None, :]
        out_ref[...] = jnp.pad(o, ((0, 0), (0, 128 - o.shape[1])))


def kernel(in_feat1, in_feat2, W1, b1, W2, b2, edge_index):
    n_nodes, f_in = in_feat1.shape
    e = edge_index.shape[1]
    npad = ((n_nodes + NBLK - 1) // NBLK) * NBLK
    feats = W1.shape[1]

    unit = NW * 8 * B
    e_pad = ((e + unit - 1) // unit) * unit
    rows = e_pad // B
    rpw = rows // NW
    sent = (jnp.arange(e_pad - e, dtype=jnp.int32) % (npad - n_nodes)
            ) + n_nodes
    src2 = jnp.concatenate([edge_index[0].astype(jnp.int32), sent]
                           ).reshape(rows, B)
    dst2 = jnp.concatenate([edge_index[1].astype(jnp.int32), sent]
                           ).reshape(rows, B)
    f1p = jnp.pad(in_feat1, ((0, npad - n_nodes), (0, 0)))
    f2p = jnp.pad(in_feat2, ((0, npad - n_nodes), (0, 0)))

    sl = npad // NS
    ones_b = jnp.ones((B,), jnp.float32)
    zeros_sl = jnp.zeros((sl,), jnp.float32)
    zrow = jnp.zeros((sl, feats), jnp.float32)

    mesh = plsc.VectorSubcoreMesh(core_axis_name="c", subcore_axis_name="s",
                                  num_cores=NC, num_subcores=NS)

    degpart = pl.kernel(
        functools.partial(_deg_body, rpw),
        out_type=jax.ShapeDtypeStruct((NC, 2, npad), jnp.float32),
        mesh=mesh,
        scratch_types=[
            pltpu.VMEM((rpw, B), jnp.int32),
            pltpu.VMEM((rpw, B), jnp.int32),
            pltpu.VMEM((B,), jnp.float32),
            pltpu.VMEM_SHARED((npad,), jnp.float32),
            pltpu.VMEM_SHARED((npad,), jnp.float32),
            pltpu.SemaphoreType.DMA,
        ],
    )(src2, dst2, ones_b, zeros_sl)

    nblocks = npad // NBLK
    y1, ndst = pl.pallas_call(
        _lin_body,
        grid=(nblocks,),
        in_specs=[
            pl.BlockSpec((NBLK, f_in), lambda i: (i, 0)),
            pl.BlockSpec((NBLK, f_in), lambda i: (i, 0)),
            pl.BlockSpec((2 * f_in, feats), lambda i: (0, 0)),
            pl.BlockSpec((NC, 2, NBLK), lambda i: (0, 0, i)),
        ],
        out_specs=[
            pl.BlockSpec((NBLK, feats), lambda i: (i, 0)),
            pl.BlockSpec((NBLK,), lambda i: (i,)),
        ],
        out_shape=[
            jax.ShapeDtypeStruct((npad, feats), jnp.float32),
            jax.ShapeDtypeStruct((npad,), jnp.float32),
        ],
    )(f1p, f2p, W1, degpart)

    src2h = src2.reshape(rows * (B // B2), B2)
    dst2h = dst2.reshape(rows * (B // B2), B2)
    aggpart, cpart = pl.kernel(
        functools.partial(_agg_body, rpw * (B // B2)),
        out_type=(
            jax.ShapeDtypeStruct((NC, npad, feats), jnp.float32),
            jax.ShapeDtypeStruct((NC, npad), jnp.float32),
        ),
        mesh=mesh,
        scratch_types=[
            pltpu.VMEM((3, 8, B2), jnp.int32),
            pltpu.VMEM((3, 8, B2), jnp.int32),
            pltpu.VMEM((NSLOT, B2, feats), jnp.float32),
            pltpu.VMEM((2, B2), jnp.float32),
            pltpu.VMEM_SHARED((npad, feats), jnp.float32),
            pltpu.VMEM_SHARED((npad,), jnp.float32),
            pltpu.VMEM_SHARED((npad,), jnp.float32),
            pltpu.SemaphoreType.DMA((3,)),
            pltpu.SemaphoreType.DMA((NSLOT,)),
            pltpu.SemaphoreType.DMA((NSLOT,)),
            pltpu.SemaphoreType.DMA((2,)),
            pltpu.SemaphoreType.DMA((2,)),
        ],
    )(src2h, dst2h, y1, ndst, zrow, zeros_sl)

    res = pl.pallas_call(
        functools.partial(_fin_body, nblocks, n_nodes),
        grid=(nblocks,),
        in_specs=[
            pl.BlockSpec((NC, NBLK, feats), lambda i: (0, i, 0)),
            pl.BlockSpec((NC, NBLK), lambda i: (0, i)),
            pl.BlockSpec((NC, 2, NBLK), lambda i: (0, 0, i)),
            pl.BlockSpec((feats, 16), lambda i: (0, 0)),
            pl.BlockSpec((feats,), lambda i: (0,)),
            pl.BlockSpec((16,), lambda i: (0,)),
        ],
        out_specs=pl.BlockSpec((1, 128), lambda i: (0, 0)),
        out_shape=jax.ShapeDtypeStruct((1, 128), jnp.float32),
        scratch_shapes=[pltpu.VMEM((1, 128), jnp.float32)],
    )(aggpart, cpart, degpart, W2, b1, b2)

    return res[0, :16]


# speedup vs baseline: 1.0198x; 1.0198x over previous
import functools

import jax
import jax.numpy as jnp
from jax import lax
from jax.experimental import pallas as pl
from jax.experimental.pallas import tpu as pltpu
from jax.experimental.pallas import tpu_sc as plsc

NC = 2
NS = 16
NW = NC * NS
B = 128
NBLK = 512


def _deg_body(rpw, src_hbm, dst_hbm, ones_hbm, zeros_hbm, out_hbm,
              sidx, didx, ones_v, dego_sp, degi_sp, sem_s):
    c = lax.axis_index("c")
    s = lax.axis_index("s")
    wid = c * NS + s
    npad = dego_sp.shape[0]
    sl = npad // NS
    pltpu.sync_copy(zeros_hbm, dego_sp.at[pl.ds(s * sl, sl)])
    pltpu.sync_copy(zeros_hbm, degi_sp.at[pl.ds(s * sl, sl)])
    pltpu.sync_copy(ones_hbm, ones_v)
    base = wid * rpw
    pltpu.sync_copy(src_hbm.at[pl.ds(base, rpw)], sidx)
    pltpu.sync_copy(dst_hbm.at[pl.ds(base, rpw)], didx)
    plsc.subcore_barrier()

    def fire(r, _):
        pltpu.async_copy(ones_v, dego_sp.at[sidx.at[r]], sem_s, add=True)
        pltpu.async_copy(ones_v, degi_sp.at[didx.at[r]], sem_s, add=True)
        return _

    lax.fori_loop(0, rpw, fire, None)

    def drain(r, _):
        pltpu.make_async_copy(ones_hbm, ones_v, sem_s).wait()
        return _

    lax.fori_loop(0, 2 * rpw, drain, None)
    plsc.subcore_barrier()
    pltpu.sync_copy(dego_sp.at[pl.ds(s * sl, sl)],
                    out_hbm.at[c, 0, pl.ds(s * sl, sl)])
    pltpu.sync_copy(degi_sp.at[pl.ds(s * sl, sl)],
                    out_hbm.at[c, 1, pl.ds(s * sl, sl)])


B2 = 64
NSLOT = 5
LEAD = NSLOT - 1


def _agg_body(rpw, src_hbm, dst_hbm, y1_hbm, ndst_hbm, zeros_hbm,
              agg_out, c_out,
              sidx, didx, rowsb, vbuf, agg_sp, c_sp, ndst_sp,
              sem_i, sem_g, sem_s, sem_v, sem_sc):
    c = lax.axis_index("c")
    s = lax.axis_index("s")
    wid = c * NS + s
    npad = c_sp.shape[0]
    sl = npad // NS
    nchunks = rpw // 8
    feats = rowsb.shape[2]

    def zrow_body(i, _):
        for k in range(feats // 16):
            rowsb[0, i, pl.ds(k * 16, 16)] = jnp.zeros((16,), jnp.float32)
        return _

    lax.fori_loop(0, B2, zrow_body, None)

    def zcopy_body(m, _):
        pltpu.sync_copy(rowsb.at[0],
                        agg_sp.at[pl.ds(s * sl + m * B2, B2), :])
        return _

    lax.fori_loop(0, sl // B2, zcopy_body, None)
    pltpu.sync_copy(zeros_hbm, c_sp.at[pl.ds(s * sl, sl)])
    pltpu.sync_copy(ndst_hbm.at[pl.ds(s * sl, sl)],
                    ndst_sp.at[pl.ds(s * sl, sl)])
    base = wid * rpw

    def stage(q):
        qs = q % 3
        pltpu.async_copy(src_hbm.at[pl.ds(base + q * 8, 8)], sidx.at[qs],
                         sem_i.at[qs])
        pltpu.async_copy(dst_hbm.at[pl.ds(base + q * 8, 8)], didx.at[qs],
                         sem_i.at[qs])

    def wait_stage(q):
        qs = q % 3
        pltpu.make_async_copy(src_hbm.at[pl.ds(0, 8)], sidx.at[qs],
                              sem_i.at[qs]).wait()
        pltpu.make_async_copy(src_hbm.at[pl.ds(0, 8)], didx.at[qs],
                              sem_i.at[qs]).wait()

    def fire_g(r, slot):
        pltpu.async_copy(y1_hbm.at[sidx.at[(r // 8) % 3, r % 8]],
                         rowsb.at[slot], sem_g.at[slot])

    def fire_v(r, slot):
        pltpu.async_copy(ndst_sp.at[didx.at[(r // 8) % 3, r % 8]],
                         vbuf.at[slot], sem_v.at[slot])

    def fire_s(r, slot):
        pltpu.async_copy(rowsb.at[slot],
                         agg_sp.at[didx.at[(r // 8) % 3, r % 8]],
                         sem_s.at[slot], add=True)

    def fire_c(r, slot):
        pltpu.async_copy(vbuf.at[slot], c_sp.at[sidx.at[(r // 8) % 3, r % 8]],
                         sem_sc.at[slot], add=True)

    def wait_rows(sem, slot):
        pltpu.make_async_copy(y1_hbm.at[pl.ds(0, B2)], rowsb.at[slot],
                              sem.at[slot]).wait()

    def wait_vals(sem, slot):
        pltpu.make_async_copy(ndst_hbm.at[pl.ds(0, B2)], vbuf.at[slot],
                              sem.at[slot]).wait()

    stage(0)
    wait_stage(0)
    stage(1)
    plsc.subcore_barrier()
    for g in range(LEAD):
        fire_g(g, g)
    fire_v(0, 0)

    def body(r, _):
        slot = r % NSLOT
        vslot = r % 2
        wait_rows(sem_g, slot)
        fire_s(r, slot)

        @pl.when(r >= 1)
        def _():
            wait_rows(sem_s, (r + LEAD) % NSLOT)

        qn = (r + LEAD) // 8

        @pl.when(((r + LEAD) % 8 == 0) & (qn <= nchunks - 1))
        def _():
            wait_stage(qn)

        @pl.when(((r + LEAD) % 8 == 0) & (qn <= nchunks - 2))
        def _():
            stage(qn + 1)

        @pl.when(r <= rpw - NSLOT)
        def _():
            fire_g(r + LEAD, (r + LEAD) % NSLOT)

        wait_vals(sem_v, vslot)
        fire_c(r, vslot)

        @pl.when(r >= 1)
        def _():
            wait_vals(sem_sc, (r + 1) % 2)

        @pl.when(r < rpw - 1)
        def _():
            fire_v(r + 1, (r + 1) % 2)

        return _

    lax.fori_loop(0, rpw, body, None)
    wait_rows(sem_s, (rpw - 1) % NSLOT)
    wait_vals(sem_sc, (rpw - 1) % 2)
    plsc.subcore_barrier()
    pltpu.sync_copy(agg_sp.at[pl.ds(s * sl, sl), :],
                    agg_out.at[c, pl.ds(s * sl, sl), :])
    pltpu.sync_copy(c_sp.at[pl.ds(s * sl, sl)],
                    c_out.at[c, pl.ds(s * sl, sl)])


def _lin_body(f1_ref, f2_ref, w1_ref, deg_ref, y1_ref, ndst_ref):
    x1 = f1_ref[...]
    x2 = f2_ref[...]
    w = w1_ref[...]
    k = x1.shape[1]
    y0 = jnp.dot(x1, w[:k, :], preferred_element_type=jnp.float32)
    y0 = y0 + jnp.dot(x2, w[k:, :], preferred_element_type=jnp.float32)
    dsrc = deg_ref[0, 0, :] + deg_ref[1, 0, :]
    ddst = deg_ref[0, 1, :] + deg_ref[1, 1, :]
    ns = lax.rsqrt(jnp.maximum(dsrc, 1.0))
    nd = lax.rsqrt(jnp.maximum(ddst, 1.0))
    n = y0.shape[0]
    ii = lax.broadcasted_iota(jnp.int32, (n, n), 0)
    jj = lax.broadcasted_iota(jnp.int32, (n, n), 1)
    dmat = jnp.where(ii == jj, ns[None, :], 0.0)
    y1_ref[...] = jnp.dot(dmat, y0, preferred_element_type=jnp.float32)
    ndst_ref[...] = nd


def _fin_body(nblocks, n_nodes, agg_ref, c_ref, deg_ref, w2_ref, b1_ref,
              b2_ref, out_ref, acc):
    i = pl.program_id(0)
    agg = agg_ref[0] + agg_ref[1]
    ddst = deg_ref[0, 1, :] + deg_ref[1, 1, :]
    dsrc = deg_ref[0, 0, :] + deg_ref[1, 0, :]
    nd = lax.rsqrt(jnp.maximum(ddst, 1.0))
    ns = lax.rsqrt(jnp.maximum(dsrc, 1.0))
    n = agg.shape[0]
    ii = lax.broadcasted_iota(jnp.int32, (n, n), 0)
    jj = lax.broadcasted_iota(jnp.int32, (n, n), 1)
    dmat = jnp.where(ii == jj, nd[None, :], 0.0)
    h = jnp.dot(dmat, agg, preferred_element_type=jnp.float32)
    h = jnp.maximum(h + b1_ref[...][None, :], 0.0)
    node = i * n + lax.broadcasted_iota(jnp.int32, (n,), 0)
    wv = (c_ref[0] + c_ref[1]) * ns
    wv = jnp.where(node < n_nodes, wv, 0.0)
    pv = jnp.dot(wv[None, :], h, preferred_element_type=jnp.float32)

    @pl.when(i == 0)
    def _():
        acc[...] = jnp.zeros_like(acc)

    acc[...] += pv

    @pl.when(i == nblocks - 1)
    def _():
        o = jnp.dot(acc[...], w2_ref[...],
                    preferred_element_type=jnp.float32)
        o = o * (1.0 / n_nodes) + b2_ref[...][None, :]
        out_ref[...] = jnp.pad(o, ((0, 0), (0, 128 - o.shape[1])))


def kernel(in_feat1, in_feat2, W1, b1, W2, b2, edge_index):
    n_nodes, f_in = in_feat1.shape
    e = edge_index.shape[1]
    npad = ((n_nodes + NBLK - 1) // NBLK) * NBLK
    feats = W1.shape[1]

    unit = NW * 8 * B
    e_pad = ((e + unit - 1) // unit) * unit
    rows = e_pad // B
    rpw = rows // NW
    sent = (jnp.arange(e_pad - e, dtype=jnp.int32) % (npad - n_nodes)
            ) + n_nodes
    src2 = jnp.concatenate([edge_index[0].astype(jnp.int32), sent]
                           ).reshape(rows, B)
    dst2 = jnp.concatenate([edge_index[1].astype(jnp.int32), sent]
                           ).reshape(rows, B)
    f1p = jnp.pad(in_feat1, ((0, npad - n_nodes), (0, 0)))
    f2p = jnp.pad(in_feat2, ((0, npad - n_nodes), (0, 0)))

    sl = npad // NS
    ones_b = jnp.ones((B,), jnp.float32)
    zeros_sl = jnp.zeros((sl,), jnp.float32)

    mesh = plsc.VectorSubcoreMesh(core_axis_name="c", subcore_axis_name="s",
                                  num_cores=NC, num_subcores=NS)

    degpart = pl.kernel(
        functools.partial(_deg_body, rpw),
        out_type=jax.ShapeDtypeStruct((NC, 2, npad), jnp.float32),
        mesh=mesh,
        scratch_types=[
            pltpu.VMEM((rpw, B), jnp.int32),
            pltpu.VMEM((rpw, B), jnp.int32),
            pltpu.VMEM((B,), jnp.float32),
            pltpu.VMEM_SHARED((npad,), jnp.float32),
            pltpu.VMEM_SHARED((npad,), jnp.float32),
            pltpu.SemaphoreType.DMA,
        ],
    )(src2, dst2, ones_b, zeros_sl)

    nblocks = npad // NBLK
    y1, ndst = pl.pallas_call(
        _lin_body,
        grid=(nblocks,),
        in_specs=[
            pl.BlockSpec((NBLK, f_in), lambda i: (i, 0)),
            pl.BlockSpec((NBLK, f_in), lambda i: (i, 0)),
            pl.BlockSpec((2 * f_in, feats), lambda i: (0, 0)),
            pl.BlockSpec((NC, 2, NBLK), lambda i: (0, 0, i)),
        ],
        out_specs=[
            pl.BlockSpec((NBLK, feats), lambda i: (i, 0)),
            pl.BlockSpec((NBLK,), lambda i: (i,)),
        ],
        out_shape=[
            jax.ShapeDtypeStruct((npad, feats), jnp.float32),
            jax.ShapeDtypeStruct((npad,), jnp.float32),
        ],
    )(f1p, f2p, W1, degpart)

    src2h = src2.reshape(rows * (B // B2), B2)
    dst2h = dst2.reshape(rows * (B // B2), B2)
    aggpart, cpart = pl.kernel(
        functools.partial(_agg_body, rpw * (B // B2)),
        out_type=(
            jax.ShapeDtypeStruct((NC, npad, feats), jnp.float32),
            jax.ShapeDtypeStruct((NC, npad), jnp.float32),
        ),
        mesh=mesh,
        scratch_types=[
            pltpu.VMEM((3, 8, B2), jnp.int32),
            pltpu.VMEM((3, 8, B2), jnp.int32),
            pltpu.VMEM((NSLOT, B2, feats), jnp.float32),
            pltpu.VMEM((2, B2), jnp.float32),
            pltpu.VMEM_SHARED((npad, feats), jnp.float32),
            pltpu.VMEM_SHARED((npad,), jnp.float32),
            pltpu.VMEM_SHARED((npad,), jnp.float32),
            pltpu.SemaphoreType.DMA((3,)),
            pltpu.SemaphoreType.DMA((NSLOT,)),
            pltpu.SemaphoreType.DMA((NSLOT,)),
            pltpu.SemaphoreType.DMA((2,)),
            pltpu.SemaphoreType.DMA((2,)),
        ],
    )(src2h, dst2h, y1, ndst, zeros_sl)

    res = pl.pallas_call(
        functools.partial(_fin_body, nblocks, n_nodes),
        grid=(nblocks,),
        in_specs=[
            pl.BlockSpec((NC, NBLK, feats), lambda i: (0, i, 0)),
            pl.BlockSpec((NC, NBLK), lambda i: (0, i)),
            pl.BlockSpec((NC, 2, NBLK), lambda i: (0, 0, i)),
            pl.BlockSpec((feats, 16), lambda i: (0, 0)),
            pl.BlockSpec((feats,), lambda i: (0,)),
            pl.BlockSpec((16,), lambda i: (0,)),
        ],
        out_specs=pl.BlockSpec((1, 128), lambda i: (0, 0)),
        out_shape=jax.ShapeDtypeStruct((1, 128), jnp.float32),
        scratch_shapes=[pltpu.VMEM((1, 128), jnp.float32)],
    )(aggpart, cpart, degpart, W2, b1, b2)

    return res[0, :16]
